# baseline (speedup 1.0000x reference)
import functools
import jax
import jax.numpy as jnp
from jax.experimental import pallas as pl

_N_NODES = 50000
_N_EDGES = 800000
_EPS = 0.05


def _mlp_body(h_ref, w0_ref, b0_ref, w1_ref, b1_ref, out_ref):
    h = h_ref[...]
    z = jnp.maximum(jnp.dot(h, w0_ref[...], preferred_element_type=jnp.float32)
                    + b0_ref[...], 0.0)
    out_ref[...] = jnp.dot(z, w1_ref[...], preferred_element_type=jnp.float32) + b1_ref[...]


def _mlp_final_body(h_ref, w0_ref, b0_ref, w1_ref, b1_ref, af_ref, out_ref):
    h = h_ref[...]
    z = jnp.maximum(jnp.dot(h, w0_ref[...], preferred_element_type=jnp.float32)
                    + b0_ref[...], 0.0)
    ev = jnp.dot(z, w1_ref[...], preferred_element_type=jnp.float32) + b1_ref[...]
    af = af_ref[...]
    diag = af == 0.0
    act = ev * (1.0 + jnp.exp(-jnp.abs(ev) / _EPS))
    v = jnp.where(diag, act, ev)
    l_vals = jnp.where(af <= 0.0, jnp.where(diag, 1.0, v), 0.0)
    u_vals = jnp.where(af >= 0.0, v, 0.0)
    out_ref[0, :, :] = l_vals
    out_ref[1, :, :] = u_vals


def _pallas_mlp(h, w0, b0, w1, b1, block_rows):
    n, k = h.shape
    kh, hid = w0.shape
    out_dim = w1.shape[1]
    assert n % block_rows == 0
    grid = (n // block_rows,)
    return pl.pallas_call(
        _mlp_body,
        grid=grid,
        in_specs=[
            pl.BlockSpec((block_rows, k), lambda i: (i, 0)),
            pl.BlockSpec((kh, hid), lambda i: (0, 0)),
            pl.BlockSpec((1, hid), lambda i: (0, 0)),
            pl.BlockSpec((hid, out_dim), lambda i: (0, 0)),
            pl.BlockSpec((1, out_dim), lambda i: (0, 0)),
        ],
        out_specs=pl.BlockSpec((block_rows, out_dim), lambda i: (i, 0)),
        out_shape=jax.ShapeDtypeStruct((n, out_dim), jnp.float32),
    )(h, w0, b0.reshape(1, -1), w1, b1.reshape(1, -1))


def _pallas_mlp_final(h, w0, b0, w1, b1, af, block_rows):
    n, k = h.shape
    kh, hid = w0.shape
    assert n % block_rows == 0
    grid = (n // block_rows,)
    return pl.pallas_call(
        _mlp_final_body,
        grid=grid,
        in_specs=[
            pl.BlockSpec((block_rows, k), lambda i: (i, 0)),
            pl.BlockSpec((kh, hid), lambda i: (0, 0)),
            pl.BlockSpec((1, hid), lambda i: (0, 0)),
            pl.BlockSpec((hid, 1), lambda i: (0, 0)),
            pl.BlockSpec((1, 1), lambda i: (0, 0)),
            pl.BlockSpec((block_rows, 1), lambda i: (i, 0)),
        ],
        out_specs=pl.BlockSpec((2, block_rows, 1), lambda i: (0, i, 0)),
        out_shape=jax.ShapeDtypeStruct((2, n, 1), jnp.float32),
    )(h, w0, b0.reshape(1, -1), w1, b1.reshape(1, -1), af)


def kernel(x, edge_attr, edge_index,
           l0_eW0, l0_eb0, l0_eW1, l0_eb1, l0_nW0, l0_nb0, l0_nW1, l0_nb1,
           l1_eW0, l1_eb0, l1_eW1, l1_eb1, l1_nW0, l1_nb0, l1_nW1, l1_nb1,
           l2_eW0, l2_eb0, l2_eW1, l2_eb1, l2_nW0, l2_nb0, l2_nW1, l2_nb1):
    row, col = edge_index[0], edge_index[1]
    af = jnp.where(row > col, -1.0, jnp.where(row < col, 1.0, 0.0)).astype(jnp.float32)[:, None]

    counts = jax.ops.segment_sum(jnp.ones((_N_EDGES,), jnp.float32), row,
                                 num_segments=_N_NODES)
    counts = jnp.maximum(counts, 1.0)[:, None]

    node_emb = x
    h0 = jnp.concatenate([node_emb[row], node_emb[col], edge_attr, af], axis=1)
    e0 = _pallas_mlp(h0, l0_eW0, l0_eb0, l0_eW1, l0_eb1, 8000)
    agg0 = jax.ops.segment_sum(e0, row, num_segments=_N_NODES) / counts
    n0 = jnp.concatenate([node_emb, agg0], axis=1)
    n0 = jnp.pad(n0, ((0, 0), (0, 7)))
    nW0_0 = jnp.pad(l0_nW0, ((0, 7), (0, 0)))
    node_emb = _pallas_mlp(n0, nW0_0, l0_nb0, l0_nW1, l0_nb1, 5000)

    h1 = jnp.concatenate([node_emb[row], node_emb[col], e0], axis=1)
    h1 = jnp.pad(h1, ((0, 0), (0, 6)))
    eW0_1 = jnp.pad(l1_eW0, ((0, 6), (0, 0)))
    e1 = _pallas_mlp(h1, eW0_1, l1_eb0, l1_eW1, l1_eb1, 8000)
    agg1 = jax.ops.segment_sum(e1, row, num_segments=_N_NODES) / counts
    n1 = jnp.concatenate([node_emb, agg1], axis=1)
    n1 = jnp.pad(n1, ((0, 0), (0, 7)))
    nW0_1 = jnp.pad(l1_nW0, ((0, 7), (0, 0)))
    node_emb = _pallas_mlp(n1, nW0_1, l1_nb0, l1_nW1, l1_nb1, 5000)

    h2 = jnp.concatenate([node_emb[row], node_emb[col], e1], axis=1)
    h2 = jnp.pad(h2, ((0, 0), (0, 6)))
    eW0_2 = jnp.pad(l2_eW0, ((0, 6), (0, 0)))
    out = _pallas_mlp_final(h2, eW0_2, l2_eb0, l2_eW1, l2_eb1, af, 8000)
    return out[:, :, 0]


# speedup vs baseline: 10.1244x; 10.1244x over previous
import jax
import jax.numpy as jnp
from jax import lax
from jax.experimental import pallas as pl
from jax.experimental.pallas import tpu as pltpu
from jax.experimental.pallas import tpu_sc as plsc

_N_NODES = 50000
_N_EDGES = 800000
_EPS = 0.05

_P = 50048
_W = 32
_CH = 1280
_CS = 128
_NCHUNK = _N_EDGES // _CH
_NW = 32
_ITERS = (_NCHUNK + _NW - 1) // _NW
_ZROWS_SC = _P // 16
_CROWS = 392

_mesh = plsc.VectorSubcoreMesh(core_axis_name="c", subcore_axis_name="s")
_sc_params = pltpu.CompilerParams(needs_layout_passes=False,
                                  use_tc_tiling_on_sc=False)


def _prep_body(x_hbm, row_hbm, col_hbm, ea_hbm, z_hbm, ar_hbm,
               s_hbm, c0_hbm, c1_hbm,
               table_v, row_v, col_v, ea_v, s_v, cnt_v, shc, idx128, idx8):
    core = lax.axis_index("c")
    sid = lax.axis_index("s")
    wid = sid * 2 + core
    pltpu.sync_copy(z_hbm, cnt_v)
    pltpu.sync_copy(x_hbm, table_v)

    def chunk_body(i, _):
        cid = i * _NW + wid

        @pl.when(cid < _NCHUNK)
        def _():
            base = cid * _CH
            pltpu.sync_copy(row_hbm.at[pl.ds(base, _CH)], row_v)
            pltpu.sync_copy(col_hbm.at[pl.ds(base, _CH)], col_v)
            pltpu.sync_copy(ea_hbm.at[pl.ds(base, _CH)], ea_v)

            def vec_body(j, _):
                r = row_v[pl.ds(j * 16, 16)]
                c = col_v[pl.ds(j * 16, 16)]
                e = ea_v[pl.ds(j * 16, 16)]
                gx = plsc.load_gather(table_v, [r])
                gc = plsc.load_gather(table_v, [c])
                af = jnp.where(r > c, -1.0, jnp.where(r < c, 1.0, 0.0))
                p4 = (lax.iota(jnp.int32, 16) + j * 16) * 4
                plsc.store_scatter(s_v, [p4], gx)
                plsc.store_scatter(s_v, [p4 + 1], gc)
                plsc.store_scatter(s_v, [p4 + 2], e)
                plsc.store_scatter(s_v, [p4 + 3], af)
                plsc.addupdate_scatter(cnt_v, [lax.shift_right_logical(r, 7),
                                               jnp.bitwise_and(r, 127)],
                                       jnp.ones((16,), jnp.float32))
                return 0

            lax.fori_loop(0, _CH // 16, vec_body, 0)
            pltpu.sync_copy(s_v, s_hbm.at[pl.ds(base * 4, _CH * 4)])

        return 0

    lax.fori_loop(0, _ITERS, chunk_body, 0)

    @pl.when(sid == 0)
    def _():
        pltpu.sync_copy(cnt_v, shc)

    plsc.subcore_barrier()

    @pl.when(sid > 0)
    def _():
        for j in range(3):
            pltpu.sync_copy(ar_hbm.at[pl.ds(j * 128, 128)], idx128)
            pltpu.sync_copy(cnt_v.at[pl.ds(j * 128, 128)], shc.at[idx128],
                            add=True)
        pltpu.sync_copy(ar_hbm.at[pl.ds(384, 8)], idx8)
        pltpu.sync_copy(cnt_v.at[pl.ds(384, 8)], shc.at[idx8], add=True)

    plsc.subcore_barrier()

    @pl.when(sid == 0)
    def _():
        @pl.when(core == 0)
        def _():
            pltpu.sync_copy(shc, c0_hbm)

        @pl.when(core == 1)
        def _():
            pltpu.sync_copy(shc, c1_hbm)


_sc_prep = pl.kernel(
    _prep_body,
    out_type=(jax.ShapeDtypeStruct((_N_EDGES * 4,), jnp.float32),
              jax.ShapeDtypeStruct((_CROWS, 128), jnp.float32),
              jax.ShapeDtypeStruct((_CROWS, 128), jnp.float32)),
    mesh=_mesh,
    compiler_params=_sc_params,
    scratch_types=[
        pltpu.VMEM((_P,), jnp.float32),
        pltpu.VMEM((_CH,), jnp.int32),
        pltpu.VMEM((_CH,), jnp.int32),
        pltpu.VMEM((_CH,), jnp.float32),
        pltpu.VMEM((_CH * 4,), jnp.float32),
        pltpu.VMEM((_CROWS, 128), jnp.float32),
        pltpu.VMEM_SHARED((_CROWS, 128), jnp.float32),
        pltpu.VMEM((128,), jnp.int32),
        pltpu.VMEM((8,), jnp.int32),
    ],
)


def _gather_body(t_hbm, row_hbm, col_hbm, s_hbm,
                 table_v, row_v, col_v, s_v):
    wid = lax.axis_index("s") * 2 + lax.axis_index("c")
    pltpu.sync_copy(t_hbm, table_v)

    def chunk_body(i, _):
        cid = i * _NW + wid

        @pl.when(cid < _NCHUNK)
        def _():
            base = cid * _CH
            pltpu.sync_copy(row_hbm.at[pl.ds(base, _CH)], row_v)
            pltpu.sync_copy(col_hbm.at[pl.ds(base, _CH)], col_v)

            def vec_body(j, _):
                r = row_v[pl.ds(j * 16, 16)]
                c = col_v[pl.ds(j * 16, 16)]
                gx = plsc.load_gather(table_v, [r])
                gc = plsc.load_gather(table_v, [c])
                p2 = (lax.iota(jnp.int32, 16) + j * 16) * 2
                plsc.store_scatter(s_v, [p2], gx)
                plsc.store_scatter(s_v, [p2 + 1], gc)
                return 0

            lax.fori_loop(0, _CH // 16, vec_body, 0)
            pltpu.sync_copy(s_v, s_hbm.at[pl.ds(base * 2, _CH * 2)])

        return 0

    lax.fori_loop(0, _ITERS, chunk_body, 0)


_sc_gather = pl.kernel(
    _gather_body,
    out_type=jax.ShapeDtypeStruct((_N_EDGES * 2,), jnp.float32),
    mesh=_mesh,
    compiler_params=_sc_params,
    scratch_types=[
        pltpu.VMEM((_P,), jnp.float32),
        pltpu.VMEM((_CH,), jnp.int32),
        pltpu.VMEM((_CH,), jnp.int32),
        pltpu.VMEM((_CH * 2,), jnp.float32),
    ],
)


def _scatter_body(e_hbm, row_hbm, z_hbm, out0_hbm, out1_hbm,
                  acc, rows_v, idx_v):
    core = lax.axis_index("c")
    sid = lax.axis_index("s")
    wid = sid * 2 + core

    pltpu.sync_copy(z_hbm, acc.at[pl.ds(sid * _ZROWS_SC, _ZROWS_SC)])
    plsc.subcore_barrier()

    def chunk_body(i, _):
        cid = i * _NW + wid

        @pl.when(cid < _N_EDGES // _CS)
        def _():
            base = cid * _CS
            pltpu.sync_copy(e_hbm.at[pl.ds(base, _CS)], rows_v)
            pltpu.sync_copy(row_hbm.at[pl.ds(base, _CS)], idx_v)
            pltpu.sync_copy(rows_v, acc.at[idx_v], add=True)

        return 0

    lax.fori_loop(0, (_N_EDGES // _CS + _NW - 1) // _NW, chunk_body, 0)
    plsc.subcore_barrier()

    src = acc.at[pl.ds(sid * _ZROWS_SC, _ZROWS_SC)]

    @pl.when(core == 0)
    def _():
        pltpu.sync_copy(src, out0_hbm.at[pl.ds(sid * _ZROWS_SC, _ZROWS_SC)])

    @pl.when(core == 1)
    def _():
        pltpu.sync_copy(src, out1_hbm.at[pl.ds(sid * _ZROWS_SC, _ZROWS_SC)])


_sc_scatter = pl.kernel(
    _scatter_body,
    out_type=(jax.ShapeDtypeStruct((_P, _W), jnp.float32),
              jax.ShapeDtypeStruct((_P, _W), jnp.float32)),
    mesh=_mesh,
    compiler_params=_sc_params,
    scratch_types=[
        pltpu.VMEM_SHARED((_P, _W), jnp.float32),
        pltpu.VMEM((_CS, _W), jnp.float32),
        pltpu.VMEM((_CS,), jnp.int32),
    ],
)


_BE = 6400
_BN = 6256


def _e0_body(s_ref, w0_ref, b0_ref, w1_ref, b1_ref, out_ref):
    h = jnp.maximum(jnp.dot(s_ref[...], w0_ref[...],
                            preferred_element_type=jnp.float32) + b0_ref[...], 0.0)
    out_ref[...] = jnp.dot(h, w1_ref[...],
                           preferred_element_type=jnp.float32) + b1_ref[...]


def _e1_body(s_ref, ep_ref, w0a_ref, w0b_ref, b0_ref, w1_ref, b1_ref, out_ref):
    h = (jnp.dot(s_ref[...], w0a_ref[...], preferred_element_type=jnp.float32)
         + jnp.dot(ep_ref[...], w0b_ref[...],
                   preferred_element_type=jnp.float32) + b0_ref[...])
    h = jnp.maximum(h, 0.0)
    out_ref[...] = jnp.dot(h, w1_ref[...],
                           preferred_element_type=jnp.float32) + b1_ref[...]


def _ef_body(s_ref, ep_ref, s0_ref, w0a_ref, w0b_ref, b0_ref, w1_ref, b1_ref,
             out_ref):
    h = (jnp.dot(s_ref[...], w0a_ref[...], preferred_element_type=jnp.float32)
         + jnp.dot(ep_ref[...], w0b_ref[...],
                   preferred_element_type=jnp.float32) + b0_ref[...])
    h = jnp.maximum(h, 0.0)
    ev = jnp.dot(h, w1_ref[...], preferred_element_type=jnp.float32) + b1_ref[...]
    af = s0_ref[:, 3:4]
    diag = af == 0.0
    act = ev * (1.0 + jnp.exp(-jnp.abs(ev) / _EPS))
    v = jnp.where(diag, act, ev)
    out_ref[0, :, :] = jnp.where(af <= 0.0, jnp.where(diag, 1.0, v), 0.0)
    out_ref[1, :, :] = jnp.where(af >= 0.0, v, 0.0)


def _node_body(x_ref, p0_ref, p1_ref, c0_ref, c1_ref,
               w0x_ref, w0a_ref, b0_ref, w1_ref, b1_ref, out_ref):
    cnt = jnp.maximum(c0_ref[...] + c1_ref[...], 1.0)
    agg = (p0_ref[...] + p1_ref[...]) / cnt
    h = jnp.maximum(x_ref[...] * w0x_ref[...]
                    + jnp.dot(agg, w0a_ref[...],
                              preferred_element_type=jnp.float32) + b0_ref[...], 0.0)
    out_ref[...] = jnp.dot(h, w1_ref[...],
                           preferred_element_type=jnp.float32) + b1_ref[...]


def _full(shape):
    return pl.BlockSpec(shape, lambda i: tuple(0 for _ in shape))


def _tc_e0(s0, w0, b0, w1, b1):
    return pl.pallas_call(
        _e0_body, grid=(_N_EDGES // _BE,),
        in_specs=[
            pl.BlockSpec((_BE, 4), lambda i: (i, 0)),
            _full((4, 32)), _full((1, 32)), _full((32, 32)), _full((1, 32)),
        ],
        out_specs=pl.BlockSpec((_BE, _W), lambda i: (i, 0)),
        out_shape=jax.ShapeDtypeStruct((_N_EDGES, _W), jnp.float32),
    )(s0, w0, b0.reshape(1, -1), w1, b1.reshape(1, -1))


def _tc_e1(s1, ep, w0a, w0b, b0, w1, b1):
    return pl.pallas_call(
        _e1_body, grid=(_N_EDGES // _BE,),
        in_specs=[
            pl.BlockSpec((_BE, 2), lambda i: (i, 0)),
            pl.BlockSpec((_BE, _W), lambda i: (i, 0)),
            _full((2, 32)), _full((32, 32)), _full((1, 32)),
            _full((32, 32)), _full((1, 32)),
        ],
        out_specs=pl.BlockSpec((_BE, _W), lambda i: (i, 0)),
        out_shape=jax.ShapeDtypeStruct((_N_EDGES, _W), jnp.float32),
    )(s1, ep, w0a, w0b, b0.reshape(1, -1), w1, b1.reshape(1, -1))


def _tc_ef(s2, ep, s0, w0a, w0b, b0, w1, b1):
    return pl.pallas_call(
        _ef_body, grid=(_N_EDGES // _BE,),
        in_specs=[
            pl.BlockSpec((_BE, 2), lambda i: (i, 0)),
            pl.BlockSpec((_BE, _W), lambda i: (i, 0)),
            pl.BlockSpec((_BE, 4), lambda i: (i, 0)),
            _full((2, 32)), _full((32, 32)), _full((1, 32)),
            _full((32, 1)), _full((1, 1)),
        ],
        out_specs=pl.BlockSpec((2, _BE, 1), lambda i: (0, i, 0)),
        out_shape=jax.ShapeDtypeStruct((2, _N_EDGES, 1), jnp.float32),
    )(s2, ep, s0, w0a, w0b, b0.reshape(1, -1), w1, b1.reshape(1, -1))


def _tc_node(x_pad, p0, p1, c0, c1, w0x, w0a, b0, w1, b1):
    return pl.pallas_call(
        _node_body, grid=(_P // _BN,),
        in_specs=[
            pl.BlockSpec((_BN, 1), lambda i: (i, 0)),
            pl.BlockSpec((_BN, _W), lambda i: (i, 0)),
            pl.BlockSpec((_BN, _W), lambda i: (i, 0)),
            pl.BlockSpec((_BN, 1), lambda i: (i, 0)),
            pl.BlockSpec((_BN, 1), lambda i: (i, 0)),
            _full((1, 32)), _full((32, 32)), _full((1, 32)),
            _full((32, 1)), _full((1, 1)),
        ],
        out_specs=pl.BlockSpec((_BN, 1), lambda i: (i, 0)),
        out_shape=jax.ShapeDtypeStruct((_P, 1), jnp.float32),
    )(x_pad, p0, p1, c0, c1, w0x, w0a, b0.reshape(1, -1), w1, b1.reshape(1, -1))


def kernel(x, edge_attr, edge_index,
           l0_eW0, l0_eb0, l0_eW1, l0_eb1, l0_nW0, l0_nb0, l0_nW1, l0_nb1,
           l1_eW0, l1_eb0, l1_eW1, l1_eb1, l1_nW0, l1_nb0, l1_nW1, l1_nb1,
           l2_eW0, l2_eb0, l2_eW1, l2_eb1, l2_nW0, l2_nb0, l2_nW1, l2_nb1):
    row = edge_index[0].astype(jnp.int32)
    col = edge_index[1].astype(jnp.int32)
    x_flat = jnp.pad(x[:, 0], (0, _P - _N_NODES))
    x_pad = x_flat[:, None]
    ea_flat = edge_attr[:, 0]
    zeros_sc = jnp.zeros((_ZROWS_SC, _W), jnp.float32)
    zeros_cnt = jnp.zeros((_CROWS, 128), jnp.float32)
    ar = jnp.arange(_CROWS, dtype=jnp.int32)

    s0, cr0, cr1 = _sc_prep(x_flat, row, col, ea_flat, zeros_cnt, ar)
    s0 = s0.reshape(_N_EDGES, 4)
    c0 = cr0.reshape(-1)[:_P, None]
    c1 = cr1.reshape(-1)[:_P, None]

    e0 = _tc_e0(s0, l0_eW0, l0_eb0, l0_eW1, l0_eb1)
    p0a, p0b = _sc_scatter(e0, row, zeros_sc)
    n1 = _tc_node(x_pad, p0a, p0b, c0, c1, l0_nW0[:1], l0_nW0[1:], l0_nb0,
                  l0_nW1, l0_nb1)

    s1 = _sc_gather(n1[:, 0], row, col).reshape(_N_EDGES, 2)
    e1 = _tc_e1(s1, e0, l1_eW0[:2], l1_eW0[2:], l1_eb0, l1_eW1, l1_eb1)
    p1a, p1b = _sc_scatter(e1, row, zeros_sc)
    n2 = _tc_node(n1, p1a, p1b, c0, c1, l1_nW0[:1], l1_nW0[1:], l1_nb0,
                  l1_nW1, l1_nb1)

    s2 = _sc_gather(n2[:, 0], row, col).reshape(_N_EDGES, 2)
    out = _tc_ef(s2, e1, s0, l2_eW0[:2], l2_eW0[2:], l2_eb0, l2_eW1, l2_eb1)
    return out[:, :, 0]


# speedup vs baseline: 10.5215x; 1.0392x over previous
import jax
import jax.numpy as jnp
from jax import lax
from jax.experimental import pallas as pl
from jax.experimental.pallas import tpu as pltpu
from jax.experimental.pallas import tpu_sc as plsc

_N_NODES = 50000
_N_EDGES = 800000
_EPS = 0.05

_P = 50048
_W = 32
_CH = 1280
_CS = 128
_NCHUNK = _N_EDGES // _CH
_NW = 32
_ITERS = (_NCHUNK + _NW - 1) // _NW
_ZROWS_SC = _P // 16
_CROWS = 392

_mesh = plsc.VectorSubcoreMesh(core_axis_name="c", subcore_axis_name="s")
_sc_params = pltpu.CompilerParams(needs_layout_passes=False,
                                  use_tc_tiling_on_sc=False)


def _prep_body(x_hbm, row_hbm, col_hbm, ea_hbm, z_hbm, ar_hbm,
               s_hbm, c0_hbm, c1_hbm,
               table_v, row_v, col_v, ea_v, s_v, cnt_v, shc, idx128, idx8):
    core = lax.axis_index("c")
    sid = lax.axis_index("s")
    wid = sid * 2 + core
    pltpu.sync_copy(z_hbm, cnt_v)
    pltpu.sync_copy(x_hbm, table_v)

    def chunk_body(i, _):
        cid = i * _NW + wid

        @pl.when(cid < _NCHUNK)
        def _():
            base = cid * _CH
            pltpu.sync_copy(row_hbm.at[pl.ds(base, _CH)], row_v)
            pltpu.sync_copy(col_hbm.at[pl.ds(base, _CH)], col_v)
            pltpu.sync_copy(ea_hbm.at[pl.ds(base, _CH)], ea_v)

            def vec_body(j, _):
                r = row_v[pl.ds(j * 16, 16)]
                c = col_v[pl.ds(j * 16, 16)]
                e = ea_v[pl.ds(j * 16, 16)]
                gx = plsc.load_gather(table_v, [r])
                gc = plsc.load_gather(table_v, [c])
                af = jnp.where(r > c, -1.0, jnp.where(r < c, 1.0, 0.0))
                p4 = (lax.iota(jnp.int32, 16) + j * 16) * 4
                plsc.store_scatter(s_v, [p4], gx)
                plsc.store_scatter(s_v, [p4 + 1], gc)
                plsc.store_scatter(s_v, [p4 + 2], e)
                plsc.store_scatter(s_v, [p4 + 3], af)
                plsc.addupdate_scatter(cnt_v, [lax.shift_right_logical(r, 7),
                                               jnp.bitwise_and(r, 127)],
                                       jnp.ones((16,), jnp.float32))
                return 0

            lax.fori_loop(0, _CH // 16, vec_body, 0)
            pltpu.sync_copy(s_v, s_hbm.at[pl.ds(base * 4, _CH * 4)])

        return 0

    lax.fori_loop(0, _ITERS, chunk_body, 0)

    @pl.when(sid == 0)
    def _():
        pltpu.sync_copy(cnt_v, shc)

    plsc.subcore_barrier()

    @pl.when(sid > 0)
    def _():
        for j in range(3):
            pltpu.sync_copy(ar_hbm.at[pl.ds(j * 128, 128)], idx128)
            pltpu.sync_copy(cnt_v.at[pl.ds(j * 128, 128)], shc.at[idx128],
                            add=True)
        pltpu.sync_copy(ar_hbm.at[pl.ds(384, 8)], idx8)
        pltpu.sync_copy(cnt_v.at[pl.ds(384, 8)], shc.at[idx8], add=True)

    plsc.subcore_barrier()

    @pl.when(sid == 0)
    def _():
        @pl.when(core == 0)
        def _():
            pltpu.sync_copy(shc, c0_hbm)

        @pl.when(core == 1)
        def _():
            pltpu.sync_copy(shc, c1_hbm)


_sc_prep = pl.kernel(
    _prep_body,
    out_type=(jax.ShapeDtypeStruct((_N_EDGES * 4,), jnp.float32),
              jax.ShapeDtypeStruct((_CROWS, 128), jnp.float32),
              jax.ShapeDtypeStruct((_CROWS, 128), jnp.float32)),
    mesh=_mesh,
    compiler_params=_sc_params,
    scratch_types=[
        pltpu.VMEM((_P,), jnp.float32),
        pltpu.VMEM((_CH,), jnp.int32),
        pltpu.VMEM((_CH,), jnp.int32),
        pltpu.VMEM((_CH,), jnp.float32),
        pltpu.VMEM((_CH * 4,), jnp.float32),
        pltpu.VMEM((_CROWS, 128), jnp.float32),
        pltpu.VMEM_SHARED((_CROWS, 128), jnp.float32),
        pltpu.VMEM((128,), jnp.int32),
        pltpu.VMEM((8,), jnp.int32),
    ],
)


def _gather_body(t_hbm, row_hbm, col_hbm, s_hbm,
                 table_v, row_v, col_v, s_v):
    wid = lax.axis_index("s") * 2 + lax.axis_index("c")
    pltpu.sync_copy(t_hbm, table_v)

    def chunk_body(i, _):
        cid = i * _NW + wid

        @pl.when(cid < _NCHUNK)
        def _():
            base = cid * _CH
            pltpu.sync_copy(row_hbm.at[pl.ds(base, _CH)], row_v)
            pltpu.sync_copy(col_hbm.at[pl.ds(base, _CH)], col_v)

            def vec_body(j, _):
                r = row_v[pl.ds(j * 16, 16)]
                c = col_v[pl.ds(j * 16, 16)]
                gx = plsc.load_gather(table_v, [r])
                gc = plsc.load_gather(table_v, [c])
                p2 = (lax.iota(jnp.int32, 16) + j * 16) * 2
                plsc.store_scatter(s_v, [p2], gx)
                plsc.store_scatter(s_v, [p2 + 1], gc)
                return 0

            lax.fori_loop(0, _CH // 16, vec_body, 0)
            pltpu.sync_copy(s_v, s_hbm.at[pl.ds(base * 2, _CH * 2)])

        return 0

    lax.fori_loop(0, _ITERS, chunk_body, 0)


_sc_gather = pl.kernel(
    _gather_body,
    out_type=jax.ShapeDtypeStruct((_N_EDGES * 2,), jnp.float32),
    mesh=_mesh,
    compiler_params=_sc_params,
    scratch_types=[
        pltpu.VMEM((_P,), jnp.float32),
        pltpu.VMEM((_CH,), jnp.int32),
        pltpu.VMEM((_CH,), jnp.int32),
        pltpu.VMEM((_CH * 2,), jnp.float32),
    ],
)


def _scatter_body(e_hbm, row_hbm, z_hbm, out0_hbm, out1_hbm,
                  acc, rows_v, idx_v):
    core = lax.axis_index("c")
    sid = lax.axis_index("s")
    wid = sid * 2 + core

    pltpu.sync_copy(z_hbm, acc.at[pl.ds(sid * _ZROWS_SC, _ZROWS_SC)])
    plsc.subcore_barrier()

    def chunk_body(i, _):
        cid = i * _NW + wid

        @pl.when(cid < _N_EDGES // _CS)
        def _():
            base = cid * _CS
            pltpu.sync_copy(e_hbm.at[pl.ds(base, _CS)], rows_v)
            pltpu.sync_copy(row_hbm.at[pl.ds(base, _CS)], idx_v)
            pltpu.sync_copy(rows_v, acc.at[idx_v], add=True)

        return 0

    lax.fori_loop(0, (_N_EDGES // _CS + _NW - 1) // _NW, chunk_body, 0)
    plsc.subcore_barrier()

    src = acc.at[pl.ds(sid * _ZROWS_SC, _ZROWS_SC)]

    @pl.when(core == 0)
    def _():
        pltpu.sync_copy(src, out0_hbm.at[pl.ds(sid * _ZROWS_SC, _ZROWS_SC)])

    @pl.when(core == 1)
    def _():
        pltpu.sync_copy(src, out1_hbm.at[pl.ds(sid * _ZROWS_SC, _ZROWS_SC)])


_sc_scatter = pl.kernel(
    _scatter_body,
    out_type=(jax.ShapeDtypeStruct((_P, _W), jnp.float32),
              jax.ShapeDtypeStruct((_P, _W), jnp.float32)),
    mesh=_mesh,
    compiler_params=_sc_params,
    scratch_types=[
        pltpu.VMEM_SHARED((_P, _W), jnp.float32),
        pltpu.VMEM((_CS, _W), jnp.float32),
        pltpu.VMEM((_CS,), jnp.int32),
    ],
)


def _finish_body(ev_hbm, row_hbm, col_hbm, out_hbm,
                 ev_v, row_v, col_v, l_v, u_v):
    wid = lax.axis_index("s") * 2 + lax.axis_index("c")

    def chunk_body(i, _):
        cid = i * _NW + wid

        @pl.when(cid < _NCHUNK)
        def _():
            base = cid * _CH
            pltpu.sync_copy(row_hbm.at[pl.ds(base, _CH)], row_v)
            pltpu.sync_copy(col_hbm.at[pl.ds(base, _CH)], col_v)
            pltpu.sync_copy(ev_hbm.at[pl.ds(base * 8, _CH * 8)], ev_v)

            def vec_body(j, _):
                r = row_v[pl.ds(j * 16, 16)]
                c = col_v[pl.ds(j * 16, 16)]
                p8 = (lax.iota(jnp.int32, 16) + j * 16) * 8
                ev = plsc.load_gather(ev_v, [p8])
                diag = r == c
                act = ev * (1.0 + jnp.exp(jnp.abs(ev) * (-1.0 / _EPS)))
                v = jnp.where(diag, act, ev)
                lv = jnp.where(r >= c, jnp.where(diag, 1.0, v), 0.0)
                uv = jnp.where(r <= c, v, 0.0)
                pos = lax.iota(jnp.int32, 16) + j * 16
                plsc.store_scatter(l_v, [pos], lv)
                plsc.store_scatter(u_v, [pos], uv)
                return 0

            lax.fori_loop(0, _CH // 16, vec_body, 0)
            pltpu.sync_copy(l_v, out_hbm.at[pl.ds(base, _CH)])
            pltpu.sync_copy(u_v, out_hbm.at[pl.ds(_N_EDGES + base, _CH)])

        return 0

    lax.fori_loop(0, _ITERS, chunk_body, 0)


_sc_finish = pl.kernel(
    _finish_body,
    out_type=jax.ShapeDtypeStruct((2 * _N_EDGES,), jnp.float32),
    mesh=_mesh,
    compiler_params=_sc_params,
    scratch_types=[
        pltpu.VMEM((_CH * 8,), jnp.float32),
        pltpu.VMEM((_CH,), jnp.int32),
        pltpu.VMEM((_CH,), jnp.int32),
        pltpu.VMEM((_CH,), jnp.float32),
        pltpu.VMEM((_CH,), jnp.float32),
    ],
)


_BE = 6400
_BN = 6256


def _e0_body(s_ref, w0_ref, b0_ref, w1_ref, b1_ref, out_ref):
    h = jnp.maximum(jnp.dot(s_ref[...], w0_ref[...],
                            preferred_element_type=jnp.float32) + b0_ref[...], 0.0)
    out_ref[...] = jnp.dot(h, w1_ref[...],
                           preferred_element_type=jnp.float32) + b1_ref[...]


def _e1_body(s_ref, ep_ref, w0a_ref, w0b_ref, b0_ref, w1_ref, b1_ref, out_ref):
    h = (jnp.dot(s_ref[...], w0a_ref[...], preferred_element_type=jnp.float32)
         + jnp.dot(ep_ref[...], w0b_ref[...],
                   preferred_element_type=jnp.float32) + b0_ref[...])
    h = jnp.maximum(h, 0.0)
    out_ref[...] = jnp.dot(h, w1_ref[...],
                           preferred_element_type=jnp.float32) + b1_ref[...]


def _e2_body(s_ref, ep_ref, w0a_ref, w0b_ref, b0_ref, w1_ref, b1_ref,
             out_ref):
    h = (jnp.dot(s_ref[...], w0a_ref[...], preferred_element_type=jnp.float32)
         + jnp.dot(ep_ref[...], w0b_ref[...],
                   preferred_element_type=jnp.float32) + b0_ref[...])
    h = jnp.maximum(h, 0.0)
    out_ref[...] = jnp.dot(h, w1_ref[...],
                           preferred_element_type=jnp.float32) + b1_ref[...]


def _node_body(x_ref, p0_ref, p1_ref, c0_ref, c1_ref,
               w0x_ref, w0a_ref, b0_ref, w1_ref, b1_ref, out_ref):
    cnt = jnp.maximum(c0_ref[...] + c1_ref[...], 1.0)
    agg = (p0_ref[...] + p1_ref[...]) / cnt
    h = jnp.maximum(x_ref[...] * w0x_ref[...]
                    + jnp.dot(agg, w0a_ref[...],
                              preferred_element_type=jnp.float32) + b0_ref[...], 0.0)
    out_ref[...] = jnp.dot(h, w1_ref[...],
                           preferred_element_type=jnp.float32) + b1_ref[...]


def _full(shape):
    return pl.BlockSpec(shape, lambda i: tuple(0 for _ in shape))


def _tc_e0(s0, w0, b0, w1, b1):
    return pl.pallas_call(
        _e0_body, grid=(_N_EDGES // _BE,),
        in_specs=[
            pl.BlockSpec((_BE, 4), lambda i: (i, 0)),
            _full((4, 32)), _full((1, 32)), _full((32, 32)), _full((1, 32)),
        ],
        out_specs=pl.BlockSpec((_BE, _W), lambda i: (i, 0)),
        out_shape=jax.ShapeDtypeStruct((_N_EDGES, _W), jnp.float32),
    )(s0, w0, b0.reshape(1, -1), w1, b1.reshape(1, -1))


def _tc_e1(s1, ep, w0a, w0b, b0, w1, b1):
    return pl.pallas_call(
        _e1_body, grid=(_N_EDGES // _BE,),
        in_specs=[
            pl.BlockSpec((_BE, 2), lambda i: (i, 0)),
            pl.BlockSpec((_BE, _W), lambda i: (i, 0)),
            _full((2, 32)), _full((32, 32)), _full((1, 32)),
            _full((32, 32)), _full((1, 32)),
        ],
        out_specs=pl.BlockSpec((_BE, _W), lambda i: (i, 0)),
        out_shape=jax.ShapeDtypeStruct((_N_EDGES, _W), jnp.float32),
    )(s1, ep, w0a, w0b, b0.reshape(1, -1), w1, b1.reshape(1, -1))


def _tc_e2(s2, ep, w0a, w0b, b0, w1, b1):
    w1p = jnp.pad(w1, ((0, 0), (0, 7)))
    b1p = jnp.pad(b1.reshape(1, -1), ((0, 0), (0, 7)))
    return pl.pallas_call(
        _e2_body, grid=(_N_EDGES // _BE,),
        in_specs=[
            pl.BlockSpec((_BE, 2), lambda i: (i, 0)),
            pl.BlockSpec((_BE, _W), lambda i: (i, 0)),
            _full((2, 32)), _full((32, 32)), _full((1, 32)),
            _full((32, 8)), _full((1, 8)),
        ],
        out_specs=pl.BlockSpec((_BE, 8), lambda i: (i, 0)),
        out_shape=jax.ShapeDtypeStruct((_N_EDGES, 8), jnp.float32),
    )(s2, ep, w0a, w0b, b0.reshape(1, -1), w1p, b1p)


def _tc_node(x_pad, p0, p1, c0, c1, w0x, w0a, b0, w1, b1):
    return pl.pallas_call(
        _node_body, grid=(_P // _BN,),
        in_specs=[
            pl.BlockSpec((_BN, 1), lambda i: (i, 0)),
            pl.BlockSpec((_BN, _W), lambda i: (i, 0)),
            pl.BlockSpec((_BN, _W), lambda i: (i, 0)),
            pl.BlockSpec((_BN, 1), lambda i: (i, 0)),
            pl.BlockSpec((_BN, 1), lambda i: (i, 0)),
            _full((1, 32)), _full((32, 32)), _full((1, 32)),
            _full((32, 1)), _full((1, 1)),
        ],
        out_specs=pl.BlockSpec((_BN, 1), lambda i: (i, 0)),
        out_shape=jax.ShapeDtypeStruct((_P, 1), jnp.float32),
    )(x_pad, p0, p1, c0, c1, w0x, w0a, b0.reshape(1, -1), w1, b1.reshape(1, -1))


def kernel(x, edge_attr, edge_index,
           l0_eW0, l0_eb0, l0_eW1, l0_eb1, l0_nW0, l0_nb0, l0_nW1, l0_nb1,
           l1_eW0, l1_eb0, l1_eW1, l1_eb1, l1_nW0, l1_nb0, l1_nW1, l1_nb1,
           l2_eW0, l2_eb0, l2_eW1, l2_eb1, l2_nW0, l2_nb0, l2_nW1, l2_nb1):
    row = edge_index[0].astype(jnp.int32)
    col = edge_index[1].astype(jnp.int32)
    x_flat = jnp.pad(x[:, 0], (0, _P - _N_NODES))
    x_pad = x_flat[:, None]
    ea_flat = edge_attr[:, 0]
    zeros_sc = jnp.zeros((_ZROWS_SC, _W), jnp.float32)
    zeros_cnt = jnp.zeros((_CROWS, 128), jnp.float32)
    ar = jnp.arange(_CROWS, dtype=jnp.int32)

    s0, cr0, cr1 = _sc_prep(x_flat, row, col, ea_flat, zeros_cnt, ar)
    s0 = s0.reshape(_N_EDGES, 4)
    c0 = cr0.reshape(-1)[:_P, None]
    c1 = cr1.reshape(-1)[:_P, None]

    e0 = _tc_e0(s0, l0_eW0, l0_eb0, l0_eW1, l0_eb1)
    p0a, p0b = _sc_scatter(e0, row, zeros_sc)
    n1 = _tc_node(x_pad, p0a, p0b, c0, c1, l0_nW0[:1], l0_nW0[1:], l0_nb0,
                  l0_nW1, l0_nb1)

    s1 = _sc_gather(n1[:, 0], row, col).reshape(_N_EDGES, 2)
    e1 = _tc_e1(s1, e0, l1_eW0[:2], l1_eW0[2:], l1_eb0, l1_eW1, l1_eb1)
    p1a, p1b = _sc_scatter(e1, row, zeros_sc)
    n2 = _tc_node(n1, p1a, p1b, c0, c1, l1_nW0[:1], l1_nW0[1:], l1_nb0,
                  l1_nW1, l1_nb1)

    s2 = _sc_gather(n2[:, 0], row, col).reshape(_N_EDGES, 2)
    ev8 = _tc_e2(s2, e1, l2_eW0[:2], l2_eW0[2:], l2_eb0, l2_eW1, l2_eb1)
    out = _sc_finish(ev8.reshape(-1), row, col)
    return out.reshape(2, _N_EDGES)


# speedup vs baseline: 10.9761x; 1.0432x over previous
import jax
import jax.numpy as jnp
from jax import lax
from jax.experimental import pallas as pl
from jax.experimental.pallas import tpu as pltpu
from jax.experimental.pallas import tpu_sc as plsc

_N_NODES = 50000
_N_EDGES = 800000
_EPS = 0.05

_P = 50048
_W = 32
_CH = 1280
_CS = 512
_NCHUNK = _N_EDGES // _CH
_NW = 32
_ITERS = (_NCHUNK + _NW - 1) // _NW
_ZROWS_SC = _P // 16
_CROWS = 392

_mesh = plsc.VectorSubcoreMesh(core_axis_name="c", subcore_axis_name="s")
_sc_params = pltpu.CompilerParams(needs_layout_passes=False,
                                  use_tc_tiling_on_sc=False)


def _prep_body(x_hbm, row_hbm, col_hbm, ea_hbm, z_hbm, ar_hbm,
               s_hbm, c0_hbm, c1_hbm,
               table_v, row_v, col_v, ea_v, s_v, cnt_v, shc, idx128, idx8):
    core = lax.axis_index("c")
    sid = lax.axis_index("s")
    wid = sid * 2 + core
    pltpu.sync_copy(z_hbm, cnt_v)
    pltpu.sync_copy(x_hbm, table_v)

    def chunk_body(i, _):
        cid = i * _NW + wid

        @pl.when(cid < _NCHUNK)
        def _():
            base = cid * _CH
            pltpu.sync_copy(row_hbm.at[pl.ds(base, _CH)], row_v)
            pltpu.sync_copy(col_hbm.at[pl.ds(base, _CH)], col_v)
            pltpu.sync_copy(ea_hbm.at[pl.ds(base, _CH)], ea_v)

            def vec_body(j, _):
                r = row_v[pl.ds(j * 16, 16)]
                c = col_v[pl.ds(j * 16, 16)]
                e = ea_v[pl.ds(j * 16, 16)]
                gx = plsc.load_gather(table_v, [r])
                gc = plsc.load_gather(table_v, [c])
                af = jnp.where(r > c, -1.0, jnp.where(r < c, 1.0, 0.0))
                p4 = (lax.iota(jnp.int32, 16) + j * 16) * 4
                plsc.store_scatter(s_v, [p4], gx)
                plsc.store_scatter(s_v, [p4 + 1], gc)
                plsc.store_scatter(s_v, [p4 + 2], e)
                plsc.store_scatter(s_v, [p4 + 3], af)
                plsc.addupdate_scatter(cnt_v, [lax.shift_right_logical(r, 7),
                                               jnp.bitwise_and(r, 127)],
                                       jnp.ones((16,), jnp.float32))
                return 0

            lax.fori_loop(0, _CH // 16, vec_body, 0)
            pltpu.sync_copy(s_v, s_hbm.at[pl.ds(base * 4, _CH * 4)])

        return 0

    lax.fori_loop(0, _ITERS, chunk_body, 0)

    @pl.when(sid == 0)
    def _():
        pltpu.sync_copy(cnt_v, shc)

    plsc.subcore_barrier()

    @pl.when(sid > 0)
    def _():
        for j in range(3):
            pltpu.sync_copy(ar_hbm.at[pl.ds(j * 128, 128)], idx128)
            pltpu.sync_copy(cnt_v.at[pl.ds(j * 128, 128)], shc.at[idx128],
                            add=True)
        pltpu.sync_copy(ar_hbm.at[pl.ds(384, 8)], idx8)
        pltpu.sync_copy(cnt_v.at[pl.ds(384, 8)], shc.at[idx8], add=True)

    plsc.subcore_barrier()

    @pl.when(sid == 0)
    def _():
        @pl.when(core == 0)
        def _():
            pltpu.sync_copy(shc, c0_hbm)

        @pl.when(core == 1)
        def _():
            pltpu.sync_copy(shc, c1_hbm)


_sc_prep = pl.kernel(
    _prep_body,
    out_type=(jax.ShapeDtypeStruct((_N_EDGES * 4,), jnp.float32),
              jax.ShapeDtypeStruct((_CROWS, 128), jnp.float32),
              jax.ShapeDtypeStruct((_CROWS, 128), jnp.float32)),
    mesh=_mesh,
    compiler_params=_sc_params,
    scratch_types=[
        pltpu.VMEM((_P,), jnp.float32),
        pltpu.VMEM((_CH,), jnp.int32),
        pltpu.VMEM((_CH,), jnp.int32),
        pltpu.VMEM((_CH,), jnp.float32),
        pltpu.VMEM((_CH * 4,), jnp.float32),
        pltpu.VMEM((_CROWS, 128), jnp.float32),
        pltpu.VMEM_SHARED((_CROWS, 128), jnp.float32),
        pltpu.VMEM((128,), jnp.int32),
        pltpu.VMEM((8,), jnp.int32),
    ],
)


def _gather_body(t_hbm, row_hbm, col_hbm, s_hbm,
                 table_v, row_v, col_v, s_v):
    wid = lax.axis_index("s") * 2 + lax.axis_index("c")
    pltpu.sync_copy(t_hbm, table_v)

    def chunk_body(i, _):
        cid = i * _NW + wid

        @pl.when(cid < _NCHUNK)
        def _():
            base = cid * _CH
            pltpu.sync_copy(row_hbm.at[pl.ds(base, _CH)], row_v)
            pltpu.sync_copy(col_hbm.at[pl.ds(base, _CH)], col_v)

            def vec_body(j, _):
                r = row_v[pl.ds(j * 16, 16)]
                c = col_v[pl.ds(j * 16, 16)]
                gx = plsc.load_gather(table_v, [r])
                gc = plsc.load_gather(table_v, [c])
                p2 = (lax.iota(jnp.int32, 16) + j * 16) * 2
                plsc.store_scatter(s_v, [p2], gx)
                plsc.store_scatter(s_v, [p2 + 1], gc)
                return 0

            lax.fori_loop(0, _CH // 16, vec_body, 0)
            pltpu.sync_copy(s_v, s_hbm.at[pl.ds(base * 2, _CH * 2)])

        return 0

    lax.fori_loop(0, _ITERS, chunk_body, 0)


_sc_gather = pl.kernel(
    _gather_body,
    out_type=jax.ShapeDtypeStruct((_N_EDGES * 2,), jnp.float32),
    mesh=_mesh,
    compiler_params=_sc_params,
    scratch_types=[
        pltpu.VMEM((_P,), jnp.float32),
        pltpu.VMEM((_CH,), jnp.int32),
        pltpu.VMEM((_CH,), jnp.int32),
        pltpu.VMEM((_CH * 2,), jnp.float32),
    ],
)


def _scatter_body(e_hbm, row_hbm, z_hbm, out0_hbm, out1_hbm,
                  acc, rows_v, idx_v0, idx_v1, idx_v2, idx_v3):
    core = lax.axis_index("c")
    sid = lax.axis_index("s")
    wid = sid * 2 + core

    pltpu.sync_copy(z_hbm, acc.at[pl.ds(sid * _ZROWS_SC, _ZROWS_SC)])
    plsc.subcore_barrier()

    def chunk_body(i, _):
        cid = i * _NW + wid

        @pl.when(cid < _N_EDGES // _CS)
        def _():
            base = cid * _CS
            pltpu.sync_copy(e_hbm.at[pl.ds(base, _CS)], rows_v)
            for j, idx_v in enumerate((idx_v0, idx_v1, idx_v2, idx_v3)):
                pltpu.sync_copy(row_hbm.at[pl.ds(base + j * 128, 128)], idx_v)
                pltpu.sync_copy(rows_v.at[pl.ds(j * 128, 128)],
                                acc.at[idx_v], add=True)

        return 0

    lax.fori_loop(0, (_N_EDGES // _CS + _NW - 1) // _NW, chunk_body, 0)
    plsc.subcore_barrier()

    src = acc.at[pl.ds(sid * _ZROWS_SC, _ZROWS_SC)]

    @pl.when(core == 0)
    def _():
        pltpu.sync_copy(src, out0_hbm.at[pl.ds(sid * _ZROWS_SC, _ZROWS_SC)])

    @pl.when(core == 1)
    def _():
        pltpu.sync_copy(src, out1_hbm.at[pl.ds(sid * _ZROWS_SC, _ZROWS_SC)])


_sc_scatter = pl.kernel(
    _scatter_body,
    out_type=(jax.ShapeDtypeStruct((_P, _W), jnp.float32),
              jax.ShapeDtypeStruct((_P, _W), jnp.float32)),
    mesh=_mesh,
    compiler_params=_sc_params,
    scratch_types=[
        pltpu.VMEM_SHARED((_P, _W), jnp.float32),
        pltpu.VMEM((_CS, _W), jnp.float32),
        pltpu.VMEM((128,), jnp.int32),
        pltpu.VMEM((128,), jnp.int32),
        pltpu.VMEM((128,), jnp.int32),
        pltpu.VMEM((128,), jnp.int32),
    ],
)


def _finish_body(ev_hbm, row_hbm, col_hbm, out_hbm,
                 ev_v, row_v, col_v, l_v, u_v):
    wid = lax.axis_index("s") * 2 + lax.axis_index("c")

    def chunk_body(i, _):
        cid = i * _NW + wid

        @pl.when(cid < _NCHUNK)
        def _():
            base = cid * _CH
            pltpu.sync_copy(row_hbm.at[pl.ds(base, _CH)], row_v)
            pltpu.sync_copy(col_hbm.at[pl.ds(base, _CH)], col_v)
            pltpu.sync_copy(ev_hbm.at[pl.ds(base * 8, _CH * 8)], ev_v)

            def vec_body(j, _):
                r = row_v[pl.ds(j * 16, 16)]
                c = col_v[pl.ds(j * 16, 16)]
                p8 = (lax.iota(jnp.int32, 16) + j * 16) * 8
                ev = plsc.load_gather(ev_v, [p8])
                diag = r == c
                act = ev * (1.0 + jnp.exp(jnp.abs(ev) * (-1.0 / _EPS)))
                v = jnp.where(diag, act, ev)
                lv = jnp.where(r >= c, jnp.where(diag, 1.0, v), 0.0)
                uv = jnp.where(r <= c, v, 0.0)
                pos = lax.iota(jnp.int32, 16) + j * 16
                plsc.store_scatter(l_v, [pos], lv)
                plsc.store_scatter(u_v, [pos], uv)
                return 0

            lax.fori_loop(0, _CH // 16, vec_body, 0)
            pltpu.sync_copy(l_v, out_hbm.at[pl.ds(base, _CH)])
            pltpu.sync_copy(u_v, out_hbm.at[pl.ds(_N_EDGES + base, _CH)])

        return 0

    lax.fori_loop(0, _ITERS, chunk_body, 0)


_sc_finish = pl.kernel(
    _finish_body,
    out_type=jax.ShapeDtypeStruct((2 * _N_EDGES,), jnp.float32),
    mesh=_mesh,
    compiler_params=_sc_params,
    scratch_types=[
        pltpu.VMEM((_CH * 8,), jnp.float32),
        pltpu.VMEM((_CH,), jnp.int32),
        pltpu.VMEM((_CH,), jnp.int32),
        pltpu.VMEM((_CH,), jnp.float32),
        pltpu.VMEM((_CH,), jnp.float32),
    ],
)


_BE = 6400
_BN = 6256


def _e0_body(s_ref, w0_ref, b0_ref, w1_ref, b1_ref, out_ref):
    h = jnp.maximum(jnp.dot(s_ref[...], w0_ref[...],
                            preferred_element_type=jnp.float32) + b0_ref[...], 0.0)
    out_ref[...] = jnp.dot(h, w1_ref[...],
                           preferred_element_type=jnp.float32) + b1_ref[...]


def _e1_body(s_ref, ep_ref, w0a_ref, w0b_ref, b0_ref, w1_ref, b1_ref, out_ref):
    h = (jnp.dot(s_ref[...], w0a_ref[...], preferred_element_type=jnp.float32)
         + jnp.dot(ep_ref[...], w0b_ref[...],
                   preferred_element_type=jnp.float32) + b0_ref[...])
    h = jnp.maximum(h, 0.0)
    out_ref[...] = jnp.dot(h, w1_ref[...],
                           preferred_element_type=jnp.float32) + b1_ref[...]


def _e2_body(s_ref, ep_ref, w0a_ref, w0b_ref, b0_ref, w1_ref, b1_ref,
             out_ref):
    h = (jnp.dot(s_ref[...], w0a_ref[...], preferred_element_type=jnp.float32)
         + jnp.dot(ep_ref[...], w0b_ref[...],
                   preferred_element_type=jnp.float32) + b0_ref[...])
    h = jnp.maximum(h, 0.0)
    out_ref[...] = jnp.dot(h, w1_ref[...],
                           preferred_element_type=jnp.float32) + b1_ref[...]


def _node_body(x_ref, p0_ref, p1_ref, c0_ref, c1_ref,
               w0x_ref, w0a_ref, b0_ref, w1_ref, b1_ref, out_ref):
    cnt = jnp.maximum(c0_ref[...] + c1_ref[...], 1.0)
    agg = (p0_ref[...] + p1_ref[...]) / cnt
    h = jnp.maximum(x_ref[...] * w0x_ref[...]
                    + jnp.dot(agg, w0a_ref[...],
                              preferred_element_type=jnp.float32) + b0_ref[...], 0.0)
    out_ref[...] = jnp.dot(h, w1_ref[...],
                           preferred_element_type=jnp.float32) + b1_ref[...]


def _full(shape):
    return pl.BlockSpec(shape, lambda i: tuple(0 for _ in shape))


def _tc_e0(s0, w0, b0, w1, b1):
    return pl.pallas_call(
        _e0_body, grid=(_N_EDGES // _BE,),
        in_specs=[
            pl.BlockSpec((_BE, 4), lambda i: (i, 0)),
            _full((4, 32)), _full((1, 32)), _full((32, 32)), _full((1, 32)),
        ],
        out_specs=pl.BlockSpec((_BE, _W), lambda i: (i, 0)),
        out_shape=jax.ShapeDtypeStruct((_N_EDGES, _W), jnp.float32),
    )(s0, w0, b0.reshape(1, -1), w1, b1.reshape(1, -1))


def _tc_e1(s1, ep, w0a, w0b, b0, w1, b1):
    return pl.pallas_call(
        _e1_body, grid=(_N_EDGES // _BE,),
        in_specs=[
            pl.BlockSpec((_BE, 2), lambda i: (i, 0)),
            pl.BlockSpec((_BE, _W), lambda i: (i, 0)),
            _full((2, 32)), _full((32, 32)), _full((1, 32)),
            _full((32, 32)), _full((1, 32)),
        ],
        out_specs=pl.BlockSpec((_BE, _W), lambda i: (i, 0)),
        out_shape=jax.ShapeDtypeStruct((_N_EDGES, _W), jnp.float32),
    )(s1, ep, w0a, w0b, b0.reshape(1, -1), w1, b1.reshape(1, -1))


def _tc_e2(s2, ep, w0a, w0b, b0, w1, b1):
    w1p = jnp.pad(w1, ((0, 0), (0, 7)))
    b1p = jnp.pad(b1.reshape(1, -1), ((0, 0), (0, 7)))
    return pl.pallas_call(
        _e2_body, grid=(_N_EDGES // _BE,),
        in_specs=[
            pl.BlockSpec((_BE, 2), lambda i: (i, 0)),
            pl.BlockSpec((_BE, _W), lambda i: (i, 0)),
            _full((2, 32)), _full((32, 32)), _full((1, 32)),
            _full((32, 8)), _full((1, 8)),
        ],
        out_specs=pl.BlockSpec((_BE, 8), lambda i: (i, 0)),
        out_shape=jax.ShapeDtypeStruct((_N_EDGES, 8), jnp.float32),
    )(s2, ep, w0a, w0b, b0.reshape(1, -1), w1p, b1p)


def _tc_node(x_pad, p0, p1, c0, c1, w0x, w0a, b0, w1, b1):
    return pl.pallas_call(
        _node_body, grid=(_P // _BN,),
        in_specs=[
            pl.BlockSpec((_BN, 1), lambda i: (i, 0)),
            pl.BlockSpec((_BN, _W), lambda i: (i, 0)),
            pl.BlockSpec((_BN, _W), lambda i: (i, 0)),
            pl.BlockSpec((_BN, 1), lambda i: (i, 0)),
            pl.BlockSpec((_BN, 1), lambda i: (i, 0)),
            _full((1, 32)), _full((32, 32)), _full((1, 32)),
            _full((32, 1)), _full((1, 1)),
        ],
        out_specs=pl.BlockSpec((_BN, 1), lambda i: (i, 0)),
        out_shape=jax.ShapeDtypeStruct((_P, 1), jnp.float32),
    )(x_pad, p0, p1, c0, c1, w0x, w0a, b0.reshape(1, -1), w1, b1.reshape(1, -1))


def kernel(x, edge_attr, edge_index,
           l0_eW0, l0_eb0, l0_eW1, l0_eb1, l0_nW0, l0_nb0, l0_nW1, l0_nb1,
           l1_eW0, l1_eb0, l1_eW1, l1_eb1, l1_nW0, l1_nb0, l1_nW1, l1_nb1,
           l2_eW0, l2_eb0, l2_eW1, l2_eb1, l2_nW0, l2_nb0, l2_nW1, l2_nb1):
    row = edge_index[0].astype(jnp.int32)
    col = edge_index[1].astype(jnp.int32)
    x_flat = jnp.pad(x[:, 0], (0, _P - _N_NODES))
    x_pad = x_flat[:, None]
    ea_flat = edge_attr[:, 0]
    zeros_sc = jnp.zeros((_ZROWS_SC, _W), jnp.float32)
    zeros_cnt = jnp.zeros((_CROWS, 128), jnp.float32)
    ar = jnp.arange(_CROWS, dtype=jnp.int32)

    s0, cr0, cr1 = _sc_prep(x_flat, row, col, ea_flat, zeros_cnt, ar)
    s0 = s0.reshape(_N_EDGES, 4)
    c0 = cr0.reshape(-1)[:_P, None]
    c1 = cr1.reshape(-1)[:_P, None]

    e0 = _tc_e0(s0, l0_eW0, l0_eb0, l0_eW1, l0_eb1)
    p0a, p0b = _sc_scatter(e0, row, zeros_sc)
    n1 = _tc_node(x_pad, p0a, p0b, c0, c1, l0_nW0[:1], l0_nW0[1:], l0_nb0,
                  l0_nW1, l0_nb1)

    s1 = _sc_gather(n1[:, 0], row, col).reshape(_N_EDGES, 2)
    e1 = _tc_e1(s1, e0, l1_eW0[:2], l1_eW0[2:], l1_eb0, l1_eW1, l1_eb1)
    p1a, p1b = _sc_scatter(e1, row, zeros_sc)
    n2 = _tc_node(n1, p1a, p1b, c0, c1, l1_nW0[:1], l1_nW0[1:], l1_nb0,
                  l1_nW1, l1_nb1)

    s2 = _sc_gather(n2[:, 0], row, col).reshape(_N_EDGES, 2)
    ev8 = _tc_e2(s2, e1, l2_eW0[:2], l2_eW0[2:], l2_eb0, l2_eW1, l2_eb1)
    out = _sc_finish(ev8.reshape(-1), row, col)
    return out.reshape(2, _N_EDGES)


# speedup vs baseline: 11.4816x; 1.0461x over previous
import jax
import jax.numpy as jnp
from jax import lax
from jax.experimental import pallas as pl
from jax.experimental.pallas import tpu as pltpu
from jax.experimental.pallas import tpu_sc as plsc

_N_NODES = 50000
_N_EDGES = 800000
_EPS = 0.05

_P = 50048
_W = 32
_CH = 1280
_CS = 512
_NCHUNK = _N_EDGES // _CH
_NW = 32
_ITERS = (_NCHUNK + _NW - 1) // _NW
_ZROWS_SC = _P // 16
_CROWS = 392

_mesh = plsc.VectorSubcoreMesh(core_axis_name="c", subcore_axis_name="s")
_sc_params = pltpu.CompilerParams(needs_layout_passes=False,
                                  use_tc_tiling_on_sc=False)


def _prep_body(x_hbm, row_hbm, col_hbm, ea_hbm, z_hbm, ar_hbm,
               s_hbm, c0_hbm, c1_hbm,
               table_v, row_v, col_v, ea_v, s_v, cnt_v, shc, idx128, idx8):
    core = lax.axis_index("c")
    sid = lax.axis_index("s")
    wid = sid * 2 + core
    pltpu.sync_copy(z_hbm, cnt_v)
    pltpu.sync_copy(x_hbm, table_v)

    def chunk_body(i, _):
        cid = i * _NW + wid

        @pl.when(cid < _NCHUNK)
        def _():
            base = cid * _CH
            pltpu.sync_copy(row_hbm.at[pl.ds(base, _CH)], row_v)
            pltpu.sync_copy(col_hbm.at[pl.ds(base, _CH)], col_v)
            pltpu.sync_copy(ea_hbm.at[pl.ds(base, _CH)], ea_v)

            def vec_body(j, _):
                r = row_v[pl.ds(j * 16, 16)]
                c = col_v[pl.ds(j * 16, 16)]
                e = ea_v[pl.ds(j * 16, 16)]
                gx = plsc.load_gather(table_v, [r])
                gc = plsc.load_gather(table_v, [c])
                af = jnp.where(r > c, -1.0, jnp.where(r < c, 1.0, 0.0))
                p4 = (lax.iota(jnp.int32, 16) + j * 16) * 4
                plsc.store_scatter(s_v, [p4], gx)
                plsc.store_scatter(s_v, [p4 + 1], gc)
                plsc.store_scatter(s_v, [p4 + 2], e)
                plsc.store_scatter(s_v, [p4 + 3], af)
                plsc.addupdate_scatter(cnt_v, [lax.shift_right_logical(r, 7),
                                               jnp.bitwise_and(r, 127)],
                                       jnp.ones((16,), jnp.float32))
                return 0

            lax.fori_loop(0, _CH // 16, vec_body, 0)
            pltpu.sync_copy(s_v, s_hbm.at[pl.ds(base * 4, _CH * 4)])

        return 0

    lax.fori_loop(0, _ITERS, chunk_body, 0)

    @pl.when(sid == 0)
    def _():
        pltpu.sync_copy(cnt_v, shc)

    plsc.subcore_barrier()

    @pl.when(sid > 0)
    def _():
        for j in range(3):
            pltpu.sync_copy(ar_hbm.at[pl.ds(j * 128, 128)], idx128)
            pltpu.sync_copy(cnt_v.at[pl.ds(j * 128, 128)], shc.at[idx128],
                            add=True)
        pltpu.sync_copy(ar_hbm.at[pl.ds(384, 8)], idx8)
        pltpu.sync_copy(cnt_v.at[pl.ds(384, 8)], shc.at[idx8], add=True)

    plsc.subcore_barrier()

    @pl.when(sid == 0)
    def _():
        @pl.when(core == 0)
        def _():
            pltpu.sync_copy(shc, c0_hbm)

        @pl.when(core == 1)
        def _():
            pltpu.sync_copy(shc, c1_hbm)


_sc_prep = pl.kernel(
    _prep_body,
    out_type=(jax.ShapeDtypeStruct((_N_EDGES * 4,), jnp.float32),
              jax.ShapeDtypeStruct((_CROWS, 128), jnp.float32),
              jax.ShapeDtypeStruct((_CROWS, 128), jnp.float32)),
    mesh=_mesh,
    compiler_params=_sc_params,
    scratch_types=[
        pltpu.VMEM((_P,), jnp.float32),
        pltpu.VMEM((_CH,), jnp.int32),
        pltpu.VMEM((_CH,), jnp.int32),
        pltpu.VMEM((_CH,), jnp.float32),
        pltpu.VMEM((_CH * 4,), jnp.float32),
        pltpu.VMEM((_CROWS, 128), jnp.float32),
        pltpu.VMEM_SHARED((_CROWS, 128), jnp.float32),
        pltpu.VMEM((128,), jnp.int32),
        pltpu.VMEM((8,), jnp.int32),
    ],
)


def _gather_body(t_hbm, row_hbm, col_hbm, s_hbm,
                 table_v, row_v, col_v, s_v):
    wid = lax.axis_index("s") * 2 + lax.axis_index("c")
    pltpu.sync_copy(t_hbm, table_v)

    def chunk_body(i, _):
        cid = i * _NW + wid

        @pl.when(cid < _NCHUNK)
        def _():
            base = cid * _CH
            pltpu.sync_copy(row_hbm.at[pl.ds(base, _CH)], row_v)
            pltpu.sync_copy(col_hbm.at[pl.ds(base, _CH)], col_v)

            def vec_body(j, _):
                r = row_v[pl.ds(j * 16, 16)]
                c = col_v[pl.ds(j * 16, 16)]
                gx = plsc.load_gather(table_v, [r])
                gc = plsc.load_gather(table_v, [c])
                p2 = (lax.iota(jnp.int32, 16) + j * 16) * 2
                plsc.store_scatter(s_v, [p2], gx)
                plsc.store_scatter(s_v, [p2 + 1], gc)
                return 0

            lax.fori_loop(0, _CH // 16, vec_body, 0)
            pltpu.sync_copy(s_v, s_hbm.at[pl.ds(base * 2, _CH * 2)])

        return 0

    lax.fori_loop(0, _ITERS, chunk_body, 0)


_sc_gather = pl.kernel(
    _gather_body,
    out_type=jax.ShapeDtypeStruct((_N_EDGES * 2,), jnp.float32),
    mesh=_mesh,
    compiler_params=_sc_params,
    scratch_types=[
        pltpu.VMEM((_P,), jnp.float32),
        pltpu.VMEM((_CH,), jnp.int32),
        pltpu.VMEM((_CH,), jnp.int32),
        pltpu.VMEM((_CH * 2,), jnp.float32),
    ],
)


def _scatter_body(e_hbm, row_hbm, z_hbm, out0_hbm, out1_hbm,
                  acc, rows_v, idx_v0, idx_v1, idx_v2, idx_v3, sem):
    core = lax.axis_index("c")
    sid = lax.axis_index("s")
    wid = sid * 2 + core

    pltpu.sync_copy(z_hbm, acc.at[pl.ds(sid * _ZROWS_SC, _ZROWS_SC)])
    plsc.subcore_barrier()

    def chunk_body(i, _):
        cid = i * _NW + wid

        @pl.when(cid < _N_EDGES // _CS)
        def _():
            base = cid * _CS
            descs = [pltpu.async_copy(e_hbm.at[pl.ds(base, _CS)], rows_v, sem)]
            for j, idx_v in enumerate((idx_v0, idx_v1, idx_v2, idx_v3)):
                descs.append(pltpu.async_copy(
                    row_hbm.at[pl.ds(base + j * 128, 128)], idx_v, sem))
            for d in descs:
                d.wait()
            for j, idx_v in enumerate((idx_v0, idx_v1, idx_v2, idx_v3)):
                pltpu.sync_copy(rows_v.at[pl.ds(j * 128, 128)],
                                acc.at[idx_v], add=True)

        return 0

    lax.fori_loop(0, (_N_EDGES // _CS + _NW - 1) // _NW, chunk_body, 0)
    plsc.subcore_barrier()

    src = acc.at[pl.ds(sid * _ZROWS_SC, _ZROWS_SC)]

    @pl.when(core == 0)
    def _():
        pltpu.sync_copy(src, out0_hbm.at[pl.ds(sid * _ZROWS_SC, _ZROWS_SC)])

    @pl.when(core == 1)
    def _():
        pltpu.sync_copy(src, out1_hbm.at[pl.ds(sid * _ZROWS_SC, _ZROWS_SC)])


_sc_scatter = pl.kernel(
    _scatter_body,
    out_type=(jax.ShapeDtypeStruct((_P, _W), jnp.float32),
              jax.ShapeDtypeStruct((_P, _W), jnp.float32)),
    mesh=_mesh,
    compiler_params=_sc_params,
    scratch_types=[
        pltpu.VMEM_SHARED((_P, _W), jnp.float32),
        pltpu.VMEM((_CS, _W), jnp.float32),
        pltpu.VMEM((128,), jnp.int32),
        pltpu.VMEM((128,), jnp.int32),
        pltpu.VMEM((128,), jnp.int32),
        pltpu.VMEM((128,), jnp.int32),
        pltpu.SemaphoreType.DMA,
    ],
)


def _finish_body(ev_hbm, row_hbm, col_hbm, out_hbm,
                 ev_v, row_v, col_v, l_v, u_v):
    wid = lax.axis_index("s") * 2 + lax.axis_index("c")

    def chunk_body(i, _):
        cid = i * _NW + wid

        @pl.when(cid < _NCHUNK)
        def _():
            base = cid * _CH
            pltpu.sync_copy(row_hbm.at[pl.ds(base, _CH)], row_v)
            pltpu.sync_copy(col_hbm.at[pl.ds(base, _CH)], col_v)
            pltpu.sync_copy(ev_hbm.at[pl.ds(base * 8, _CH * 8)], ev_v)

            def vec_body(j, _):
                r = row_v[pl.ds(j * 16, 16)]
                c = col_v[pl.ds(j * 16, 16)]
                p8 = (lax.iota(jnp.int32, 16) + j * 16) * 8
                ev = plsc.load_gather(ev_v, [p8])
                diag = r == c
                act = ev * (1.0 + jnp.exp(jnp.abs(ev) * (-1.0 / _EPS)))
                v = jnp.where(diag, act, ev)
                lv = jnp.where(r >= c, jnp.where(diag, 1.0, v), 0.0)
                uv = jnp.where(r <= c, v, 0.0)
                pos = lax.iota(jnp.int32, 16) + j * 16
                plsc.store_scatter(l_v, [pos], lv)
                plsc.store_scatter(u_v, [pos], uv)
                return 0

            lax.fori_loop(0, _CH // 16, vec_body, 0)
            pltpu.sync_copy(l_v, out_hbm.at[pl.ds(base, _CH)])
            pltpu.sync_copy(u_v, out_hbm.at[pl.ds(_N_EDGES + base, _CH)])

        return 0

    lax.fori_loop(0, _ITERS, chunk_body, 0)


_sc_finish = pl.kernel(
    _finish_body,
    out_type=jax.ShapeDtypeStruct((2 * _N_EDGES,), jnp.float32),
    mesh=_mesh,
    compiler_params=_sc_params,
    scratch_types=[
        pltpu.VMEM((_CH * 8,), jnp.float32),
        pltpu.VMEM((_CH,), jnp.int32),
        pltpu.VMEM((_CH,), jnp.int32),
        pltpu.VMEM((_CH,), jnp.float32),
        pltpu.VMEM((_CH,), jnp.float32),
    ],
)


_BE = 6400
_BN = 6256


def _e0_body(s_ref, w0_ref, b0_ref, w1_ref, b1_ref, out_ref):
    h = jnp.maximum(jnp.dot(s_ref[...], w0_ref[...],
                            preferred_element_type=jnp.float32) + b0_ref[...], 0.0)
    out_ref[...] = jnp.dot(h, w1_ref[...],
                           preferred_element_type=jnp.float32) + b1_ref[...]


def _e1_body(s_ref, ep_ref, w0a_ref, w0b_ref, b0_ref, w1_ref, b1_ref, out_ref):
    h = (jnp.dot(s_ref[...], w0a_ref[...], preferred_element_type=jnp.float32)
         + jnp.dot(ep_ref[...], w0b_ref[...],
                   preferred_element_type=jnp.float32) + b0_ref[...])
    h = jnp.maximum(h, 0.0)
    out_ref[...] = jnp.dot(h, w1_ref[...],
                           preferred_element_type=jnp.float32) + b1_ref[...]


def _e2_body(s_ref, ep_ref, w0a_ref, w0b_ref, b0_ref, w1_ref, b1_ref,
             out_ref):
    h = (jnp.dot(s_ref[...], w0a_ref[...], preferred_element_type=jnp.float32)
         + jnp.dot(ep_ref[...], w0b_ref[...],
                   preferred_element_type=jnp.float32) + b0_ref[...])
    h = jnp.maximum(h, 0.0)
    out_ref[...] = jnp.dot(h, w1_ref[...],
                           preferred_element_type=jnp.float32) + b1_ref[...]


def _node_body(x_ref, p0_ref, p1_ref, c0_ref, c1_ref,
               w0x_ref, w0a_ref, b0_ref, w1_ref, b1_ref, out_ref):
    cnt = jnp.maximum(c0_ref[...] + c1_ref[...], 1.0)
    agg = (p0_ref[...] + p1_ref[...]) / cnt
    h = jnp.maximum(x_ref[...] * w0x_ref[...]
                    + jnp.dot(agg, w0a_ref[...],
                              preferred_element_type=jnp.float32) + b0_ref[...], 0.0)
    out_ref[...] = jnp.dot(h, w1_ref[...],
                           preferred_element_type=jnp.float32) + b1_ref[...]


def _full(shape):
    return pl.BlockSpec(shape, lambda i: tuple(0 for _ in shape))


def _tc_e0(s0, w0, b0, w1, b1):
    return pl.pallas_call(
        _e0_body, grid=(_N_EDGES // _BE,),
        in_specs=[
            pl.BlockSpec((_BE, 4), lambda i: (i, 0)),
            _full((4, 32)), _full((1, 32)), _full((32, 32)), _full((1, 32)),
        ],
        out_specs=pl.BlockSpec((_BE, _W), lambda i: (i, 0)),
        out_shape=jax.ShapeDtypeStruct((_N_EDGES, _W), jnp.float32),
    )(s0, w0, b0.reshape(1, -1), w1, b1.reshape(1, -1))


def _tc_e1(s1, ep, w0a, w0b, b0, w1, b1):
    return pl.pallas_call(
        _e1_body, grid=(_N_EDGES // _BE,),
        in_specs=[
            pl.BlockSpec((_BE, 2), lambda i: (i, 0)),
            pl.BlockSpec((_BE, _W), lambda i: (i, 0)),
            _full((2, 32)), _full((32, 32)), _full((1, 32)),
            _full((32, 32)), _full((1, 32)),
        ],
        out_specs=pl.BlockSpec((_BE, _W), lambda i: (i, 0)),
        out_shape=jax.ShapeDtypeStruct((_N_EDGES, _W), jnp.float32),
    )(s1, ep, w0a, w0b, b0.reshape(1, -1), w1, b1.reshape(1, -1))


def _tc_e2(s2, ep, w0a, w0b, b0, w1, b1):
    w1p = jnp.pad(w1, ((0, 0), (0, 7)))
    b1p = jnp.pad(b1.reshape(1, -1), ((0, 0), (0, 7)))
    return pl.pallas_call(
        _e2_body, grid=(_N_EDGES // _BE,),
        in_specs=[
            pl.BlockSpec((_BE, 2), lambda i: (i, 0)),
            pl.BlockSpec((_BE, _W), lambda i: (i, 0)),
            _full((2, 32)), _full((32, 32)), _full((1, 32)),
            _full((32, 8)), _full((1, 8)),
        ],
        out_specs=pl.BlockSpec((_BE, 8), lambda i: (i, 0)),
        out_shape=jax.ShapeDtypeStruct((_N_EDGES, 8), jnp.float32),
    )(s2, ep, w0a, w0b, b0.reshape(1, -1), w1p, b1p)


def _tc_node(x_pad, p0, p1, c0, c1, w0x, w0a, b0, w1, b1):
    return pl.pallas_call(
        _node_body, grid=(_P // _BN,),
        in_specs=[
            pl.BlockSpec((_BN, 1), lambda i: (i, 0)),
            pl.BlockSpec((_BN, _W), lambda i: (i, 0)),
            pl.BlockSpec((_BN, _W), lambda i: (i, 0)),
            pl.BlockSpec((_BN, 1), lambda i: (i, 0)),
            pl.BlockSpec((_BN, 1), lambda i: (i, 0)),
            _full((1, 32)), _full((32, 32)), _full((1, 32)),
            _full((32, 1)), _full((1, 1)),
        ],
        out_specs=pl.BlockSpec((_BN, 1), lambda i: (i, 0)),
        out_shape=jax.ShapeDtypeStruct((_P, 1), jnp.float32),
    )(x_pad, p0, p1, c0, c1, w0x, w0a, b0.reshape(1, -1), w1, b1.reshape(1, -1))


def kernel(x, edge_attr, edge_index,
           l0_eW0, l0_eb0, l0_eW1, l0_eb1, l0_nW0, l0_nb0, l0_nW1, l0_nb1,
           l1_eW0, l1_eb0, l1_eW1, l1_eb1, l1_nW0, l1_nb0, l1_nW1, l1_nb1,
           l2_eW0, l2_eb0, l2_eW1, l2_eb1, l2_nW0, l2_nb0, l2_nW1, l2_nb1):
    row = edge_index[0].astype(jnp.int32)
    col = edge_index[1].astype(jnp.int32)
    x_flat = jnp.pad(x[:, 0], (0, _P - _N_NODES))
    x_pad = x_flat[:, None]
    ea_flat = edge_attr[:, 0]
    zeros_sc = jnp.zeros((_ZROWS_SC, _W), jnp.float32)
    zeros_cnt = jnp.zeros((_CROWS, 128), jnp.float32)
    ar = jnp.arange(_CROWS, dtype=jnp.int32)

    s0, cr0, cr1 = _sc_prep(x_flat, row, col, ea_flat, zeros_cnt, ar)
    s0 = s0.reshape(_N_EDGES, 4)
    c0 = cr0.reshape(-1)[:_P, None]
    c1 = cr1.reshape(-1)[:_P, None]

    e0 = _tc_e0(s0, l0_eW0, l0_eb0, l0_eW1, l0_eb1)
    p0a, p0b = _sc_scatter(e0, row, zeros_sc)
    n1 = _tc_node(x_pad, p0a, p0b, c0, c1, l0_nW0[:1], l0_nW0[1:], l0_nb0,
                  l0_nW1, l0_nb1)

    s1 = _sc_gather(n1[:, 0], row, col).reshape(_N_EDGES, 2)
    e1 = _tc_e1(s1, e0, l1_eW0[:2], l1_eW0[2:], l1_eb0, l1_eW1, l1_eb1)
    p1a, p1b = _sc_scatter(e1, row, zeros_sc)
    n2 = _tc_node(n1, p1a, p1b, c0, c1, l1_nW0[:1], l1_nW0[1:], l1_nb0,
                  l1_nW1, l1_nb1)

    s2 = _sc_gather(n2[:, 0], row, col).reshape(_N_EDGES, 2)
    ev8 = _tc_e2(s2, e1, l2_eW0[:2], l2_eW0[2:], l2_eb0, l2_eW1, l2_eb1)
    out = _sc_finish(ev8.reshape(-1), row, col)
    return out.reshape(2, _N_EDGES)


# speedup vs baseline: 11.6566x; 1.0152x over previous
import jax
import jax.numpy as jnp
from jax import lax
from jax.experimental import pallas as pl
from jax.experimental.pallas import tpu as pltpu
from jax.experimental.pallas import tpu_sc as plsc

_N_NODES = 50000
_N_EDGES = 800000
_EPS = 0.05

_P = 50048
_W = 32
_CH = 1280
_CS = 512
_NCHUNK = _N_EDGES // _CH
_NW = 32
_ITERS = (_NCHUNK + _NW - 1) // _NW
_ZROWS_SC = _P // 16
_CROWS = 392

_mesh = plsc.VectorSubcoreMesh(core_axis_name="c", subcore_axis_name="s")
_sc_params = pltpu.CompilerParams(needs_layout_passes=False,
                                  use_tc_tiling_on_sc=False)


def _prep_body(x_hbm, row_hbm, col_hbm, ea_hbm, z_hbm, ar_hbm,
               s_hbm, c0_hbm, c1_hbm,
               table_v, row_v, col_v, ea_v, s_v, cnt_v, shc, idx128, idx8, sem):
    core = lax.axis_index("c")
    sid = lax.axis_index("s")
    wid = sid * 2 + core
    pltpu.sync_copy(z_hbm, cnt_v)
    pltpu.sync_copy(x_hbm, table_v)

    def chunk_body(i, _):
        cid = i * _NW + wid

        @pl.when(cid < _NCHUNK)
        def _():
            base = cid * _CH
            descs = [pltpu.async_copy(row_hbm.at[pl.ds(base, _CH)], row_v, sem),
                     pltpu.async_copy(col_hbm.at[pl.ds(base, _CH)], col_v, sem),
                     pltpu.async_copy(ea_hbm.at[pl.ds(base, _CH)], ea_v, sem)]
            for d in descs:
                d.wait()

            def vec_body(j, _):
                r = row_v[pl.ds(j * 16, 16)]
                c = col_v[pl.ds(j * 16, 16)]
                e = ea_v[pl.ds(j * 16, 16)]
                gx = plsc.load_gather(table_v, [r])
                gc = plsc.load_gather(table_v, [c])
                af = jnp.where(r > c, -1.0, jnp.where(r < c, 1.0, 0.0))
                p4 = (lax.iota(jnp.int32, 16) + j * 16) * 4
                plsc.store_scatter(s_v, [p4], gx)
                plsc.store_scatter(s_v, [p4 + 1], gc)
                plsc.store_scatter(s_v, [p4 + 2], e)
                plsc.store_scatter(s_v, [p4 + 3], af)
                plsc.addupdate_scatter(cnt_v, [lax.shift_right_logical(r, 7),
                                               jnp.bitwise_and(r, 127)],
                                       jnp.ones((16,), jnp.float32))
                return 0

            lax.fori_loop(0, _CH // 16, vec_body, 0)
            pltpu.sync_copy(s_v, s_hbm.at[pl.ds(base * 4, _CH * 4)])

        return 0

    lax.fori_loop(0, _ITERS, chunk_body, 0)

    @pl.when(sid == 0)
    def _():
        pltpu.sync_copy(cnt_v, shc)

    plsc.subcore_barrier()

    @pl.when(sid > 0)
    def _():
        for j in range(3):
            pltpu.sync_copy(ar_hbm.at[pl.ds(j * 128, 128)], idx128)
            pltpu.sync_copy(cnt_v.at[pl.ds(j * 128, 128)], shc.at[idx128],
                            add=True)
        pltpu.sync_copy(ar_hbm.at[pl.ds(384, 8)], idx8)
        pltpu.sync_copy(cnt_v.at[pl.ds(384, 8)], shc.at[idx8], add=True)

    plsc.subcore_barrier()

    @pl.when(sid == 0)
    def _():
        @pl.when(core == 0)
        def _():
            pltpu.sync_copy(shc, c0_hbm)

        @pl.when(core == 1)
        def _():
            pltpu.sync_copy(shc, c1_hbm)


_sc_prep = pl.kernel(
    _prep_body,
    out_type=(jax.ShapeDtypeStruct((_N_EDGES * 4,), jnp.float32),
              jax.ShapeDtypeStruct((_CROWS, 128), jnp.float32),
              jax.ShapeDtypeStruct((_CROWS, 128), jnp.float32)),
    mesh=_mesh,
    compiler_params=_sc_params,
    scratch_types=[
        pltpu.VMEM((_P,), jnp.float32),
        pltpu.VMEM((_CH,), jnp.int32),
        pltpu.VMEM((_CH,), jnp.int32),
        pltpu.VMEM((_CH,), jnp.float32),
        pltpu.VMEM((_CH * 4,), jnp.float32),
        pltpu.VMEM((_CROWS, 128), jnp.float32),
        pltpu.VMEM_SHARED((_CROWS, 128), jnp.float32),
        pltpu.VMEM((128,), jnp.int32),
        pltpu.VMEM((8,), jnp.int32),
        pltpu.SemaphoreType.DMA,
    ],
)


def _gather_body(t_hbm, row_hbm, col_hbm, s_hbm,
                 table_v, row_v, col_v, s_v, sem):
    wid = lax.axis_index("s") * 2 + lax.axis_index("c")
    pltpu.sync_copy(t_hbm, table_v)

    def chunk_body(i, _):
        cid = i * _NW + wid

        @pl.when(cid < _NCHUNK)
        def _():
            base = cid * _CH
            descs = [pltpu.async_copy(row_hbm.at[pl.ds(base, _CH)], row_v, sem),
                     pltpu.async_copy(col_hbm.at[pl.ds(base, _CH)], col_v, sem)]
            for d in descs:
                d.wait()

            def vec_body(j, _):
                r = row_v[pl.ds(j * 16, 16)]
                c = col_v[pl.ds(j * 16, 16)]
                gx = plsc.load_gather(table_v, [r])
                gc = plsc.load_gather(table_v, [c])
                p2 = (lax.iota(jnp.int32, 16) + j * 16) * 2
                plsc.store_scatter(s_v, [p2], gx)
                plsc.store_scatter(s_v, [p2 + 1], gc)
                return 0

            lax.fori_loop(0, _CH // 16, vec_body, 0)
            pltpu.sync_copy(s_v, s_hbm.at[pl.ds(base * 2, _CH * 2)])

        return 0

    lax.fori_loop(0, _ITERS, chunk_body, 0)


_sc_gather = pl.kernel(
    _gather_body,
    out_type=jax.ShapeDtypeStruct((_N_EDGES * 2,), jnp.float32),
    mesh=_mesh,
    compiler_params=_sc_params,
    scratch_types=[
        pltpu.VMEM((_P,), jnp.float32),
        pltpu.VMEM((_CH,), jnp.int32),
        pltpu.VMEM((_CH,), jnp.int32),
        pltpu.VMEM((_CH * 2,), jnp.float32),
        pltpu.SemaphoreType.DMA,
    ],
)


def _scatter_body(e_hbm, row_hbm, z_hbm, out0_hbm, out1_hbm,
                  acc, rows_v, idx_v0, idx_v1, idx_v2, idx_v3, sem):
    core = lax.axis_index("c")
    sid = lax.axis_index("s")
    wid = sid * 2 + core

    pltpu.sync_copy(z_hbm, acc.at[pl.ds(sid * _ZROWS_SC, _ZROWS_SC)])
    plsc.subcore_barrier()

    def chunk_body(i, _):
        cid = i * _NW + wid

        @pl.when(cid < _N_EDGES // _CS)
        def _():
            base = cid * _CS
            descs = [pltpu.async_copy(e_hbm.at[pl.ds(base, _CS)], rows_v, sem)]
            for j, idx_v in enumerate((idx_v0, idx_v1, idx_v2, idx_v3)):
                descs.append(pltpu.async_copy(
                    row_hbm.at[pl.ds(base + j * 128, 128)], idx_v, sem))
            for d in descs:
                d.wait()
            for j, idx_v in enumerate((idx_v0, idx_v1, idx_v2, idx_v3)):
                pltpu.sync_copy(rows_v.at[pl.ds(j * 128, 128)],
                                acc.at[idx_v], add=True)

        return 0

    lax.fori_loop(0, (_N_EDGES // _CS + _NW - 1) // _NW, chunk_body, 0)
    plsc.subcore_barrier()

    src = acc.at[pl.ds(sid * _ZROWS_SC, _ZROWS_SC)]

    @pl.when(core == 0)
    def _():
        pltpu.sync_copy(src, out0_hbm.at[pl.ds(sid * _ZROWS_SC, _ZROWS_SC)])

    @pl.when(core == 1)
    def _():
        pltpu.sync_copy(src, out1_hbm.at[pl.ds(sid * _ZROWS_SC, _ZROWS_SC)])


_sc_scatter = pl.kernel(
    _scatter_body,
    out_type=(jax.ShapeDtypeStruct((_P, _W), jnp.float32),
              jax.ShapeDtypeStruct((_P, _W), jnp.float32)),
    mesh=_mesh,
    compiler_params=_sc_params,
    scratch_types=[
        pltpu.VMEM_SHARED((_P, _W), jnp.float32),
        pltpu.VMEM((_CS, _W), jnp.float32),
        pltpu.VMEM((128,), jnp.int32),
        pltpu.VMEM((128,), jnp.int32),
        pltpu.VMEM((128,), jnp.int32),
        pltpu.VMEM((128,), jnp.int32),
        pltpu.SemaphoreType.DMA,
    ],
)


def _finish_body(ev_hbm, row_hbm, col_hbm, out_hbm,
                 ev_v, row_v, col_v, l_v, u_v, sem):
    wid = lax.axis_index("s") * 2 + lax.axis_index("c")

    def chunk_body(i, _):
        cid = i * _NW + wid

        @pl.when(cid < _NCHUNK)
        def _():
            base = cid * _CH
            descs = [pltpu.async_copy(row_hbm.at[pl.ds(base, _CH)], row_v, sem),
                     pltpu.async_copy(col_hbm.at[pl.ds(base, _CH)], col_v, sem),
                     pltpu.async_copy(ev_hbm.at[pl.ds(base * 8, _CH * 8)],
                                      ev_v, sem)]
            for d in descs:
                d.wait()

            def vec_body(j, _):
                r = row_v[pl.ds(j * 16, 16)]
                c = col_v[pl.ds(j * 16, 16)]
                p8 = (lax.iota(jnp.int32, 16) + j * 16) * 8
                ev = plsc.load_gather(ev_v, [p8])
                diag = r == c
                act = ev * (1.0 + jnp.exp(jnp.abs(ev) * (-1.0 / _EPS)))
                v = jnp.where(diag, act, ev)
                lv = jnp.where(r >= c, jnp.where(diag, 1.0, v), 0.0)
                uv = jnp.where(r <= c, v, 0.0)
                pos = lax.iota(jnp.int32, 16) + j * 16
                plsc.store_scatter(l_v, [pos], lv)
                plsc.store_scatter(u_v, [pos], uv)
                return 0

            lax.fori_loop(0, _CH // 16, vec_body, 0)
            pltpu.sync_copy(l_v, out_hbm.at[pl.ds(base, _CH)])
            pltpu.sync_copy(u_v, out_hbm.at[pl.ds(_N_EDGES + base, _CH)])

        return 0

    lax.fori_loop(0, _ITERS, chunk_body, 0)


_sc_finish = pl.kernel(
    _finish_body,
    out_type=jax.ShapeDtypeStruct((2 * _N_EDGES,), jnp.float32),
    mesh=_mesh,
    compiler_params=_sc_params,
    scratch_types=[
        pltpu.VMEM((_CH * 8,), jnp.float32),
        pltpu.VMEM((_CH,), jnp.int32),
        pltpu.VMEM((_CH,), jnp.int32),
        pltpu.VMEM((_CH,), jnp.float32),
        pltpu.VMEM((_CH,), jnp.float32),
        pltpu.SemaphoreType.DMA,
    ],
)


_BE = 6400
_BN = 6256


def _e0_body(s_ref, w0_ref, b0_ref, w1_ref, b1_ref, out_ref):
    h = jnp.maximum(jnp.dot(s_ref[...], w0_ref[...],
                            preferred_element_type=jnp.float32) + b0_ref[...], 0.0)
    out_ref[...] = jnp.dot(h, w1_ref[...],
                           preferred_element_type=jnp.float32) + b1_ref[...]


def _e1_body(s_ref, ep_ref, w0a_ref, w0b_ref, b0_ref, w1_ref, b1_ref, out_ref):
    h = (jnp.dot(s_ref[...], w0a_ref[...], preferred_element_type=jnp.float32)
         + jnp.dot(ep_ref[...], w0b_ref[...],
                   preferred_element_type=jnp.float32) + b0_ref[...])
    h = jnp.maximum(h, 0.0)
    out_ref[...] = jnp.dot(h, w1_ref[...],
                           preferred_element_type=jnp.float32) + b1_ref[...]


def _e2_body(s_ref, ep_ref, w0a_ref, w0b_ref, b0_ref, w1_ref, b1_ref,
             out_ref):
    h = (jnp.dot(s_ref[...], w0a_ref[...], preferred_element_type=jnp.float32)
         + jnp.dot(ep_ref[...], w0b_ref[...],
                   preferred_element_type=jnp.float32) + b0_ref[...])
    h = jnp.maximum(h, 0.0)
    out_ref[...] = jnp.dot(h, w1_ref[...],
                           preferred_element_type=jnp.float32) + b1_ref[...]


def _node_body(x_ref, p0_ref, p1_ref, c0_ref, c1_ref,
               w0x_ref, w0a_ref, b0_ref, w1_ref, b1_ref, out_ref):
    cnt = jnp.maximum(c0_ref[...] + c1_ref[...], 1.0)
    agg = (p0_ref[...] + p1_ref[...]) / cnt
    h = jnp.maximum(x_ref[...] * w0x_ref[...]
                    + jnp.dot(agg, w0a_ref[...],
                              preferred_element_type=jnp.float32) + b0_ref[...], 0.0)
    out_ref[...] = jnp.dot(h, w1_ref[...],
                           preferred_element_type=jnp.float32) + b1_ref[...]


def _full(shape):
    return pl.BlockSpec(shape, lambda i: tuple(0 for _ in shape))


def _tc_e0(s0, w0, b0, w1, b1):
    return pl.pallas_call(
        _e0_body, grid=(_N_EDGES // _BE,),
        in_specs=[
            pl.BlockSpec((_BE, 4), lambda i: (i, 0)),
            _full((4, 32)), _full((1, 32)), _full((32, 32)), _full((1, 32)),
        ],
        out_specs=pl.BlockSpec((_BE, _W), lambda i: (i, 0)),
        out_shape=jax.ShapeDtypeStruct((_N_EDGES, _W), jnp.float32),
    )(s0, w0, b0.reshape(1, -1), w1, b1.reshape(1, -1))


def _tc_e1(s1, ep, w0a, w0b, b0, w1, b1):
    return pl.pallas_call(
        _e1_body, grid=(_N_EDGES // _BE,),
        in_specs=[
            pl.BlockSpec((_BE, 2), lambda i: (i, 0)),
            pl.BlockSpec((_BE, _W), lambda i: (i, 0)),
            _full((2, 32)), _full((32, 32)), _full((1, 32)),
            _full((32, 32)), _full((1, 32)),
        ],
        out_specs=pl.BlockSpec((_BE, _W), lambda i: (i, 0)),
        out_shape=jax.ShapeDtypeStruct((_N_EDGES, _W), jnp.float32),
    )(s1, ep, w0a, w0b, b0.reshape(1, -1), w1, b1.reshape(1, -1))


def _tc_e2(s2, ep, w0a, w0b, b0, w1, b1):
    w1p = jnp.pad(w1, ((0, 0), (0, 7)))
    b1p = jnp.pad(b1.reshape(1, -1), ((0, 0), (0, 7)))
    return pl.pallas_call(
        _e2_body, grid=(_N_EDGES // _BE,),
        in_specs=[
            pl.BlockSpec((_BE, 2), lambda i: (i, 0)),
            pl.BlockSpec((_BE, _W), lambda i: (i, 0)),
            _full((2, 32)), _full((32, 32)), _full((1, 32)),
            _full((32, 8)), _full((1, 8)),
        ],
        out_specs=pl.BlockSpec((_BE, 8), lambda i: (i, 0)),
        out_shape=jax.ShapeDtypeStruct((_N_EDGES, 8), jnp.float32),
    )(s2, ep, w0a, w0b, b0.reshape(1, -1), w1p, b1p)


def _tc_node(x_pad, p0, p1, c0, c1, w0x, w0a, b0, w1, b1):
    return pl.pallas_call(
        _node_body, grid=(_P // _BN,),
        in_specs=[
            pl.BlockSpec((_BN, 1), lambda i: (i, 0)),
            pl.BlockSpec((_BN, _W), lambda i: (i, 0)),
            pl.BlockSpec((_BN, _W), lambda i: (i, 0)),
            pl.BlockSpec((_BN, 1), lambda i: (i, 0)),
            pl.BlockSpec((_BN, 1), lambda i: (i, 0)),
            _full((1, 32)), _full((32, 32)), _full((1, 32)),
            _full((32, 1)), _full((1, 1)),
        ],
        out_specs=pl.BlockSpec((_BN, 1), lambda i: (i, 0)),
        out_shape=jax.ShapeDtypeStruct((_P, 1), jnp.float32),
    )(x_pad, p0, p1, c0, c1, w0x, w0a, b0.reshape(1, -1), w1, b1.reshape(1, -1))


def kernel(x, edge_attr, edge_index,
           l0_eW0, l0_eb0, l0_eW1, l0_eb1, l0_nW0, l0_nb0, l0_nW1, l0_nb1,
           l1_eW0, l1_eb0, l1_eW1, l1_eb1, l1_nW0, l1_nb0, l1_nW1, l1_nb1,
           l2_eW0, l2_eb0, l2_eW1, l2_eb1, l2_nW0, l2_nb0, l2_nW1, l2_nb1):
    row = edge_index[0].astype(jnp.int32)
    col = edge_index[1].astype(jnp.int32)
    x_flat = jnp.pad(x[:, 0], (0, _P - _N_NODES))
    x_pad = x_flat[:, None]
    ea_flat = edge_attr[:, 0]
    zeros_sc = jnp.zeros((_ZROWS_SC, _W), jnp.float32)
    zeros_cnt = jnp.zeros((_CROWS, 128), jnp.float32)
    ar = jnp.arange(_CROWS, dtype=jnp.int32)

    s0, cr0, cr1 = _sc_prep(x_flat, row, col, ea_flat, zeros_cnt, ar)
    s0 = s0.reshape(_N_EDGES, 4)
    c0 = cr0.reshape(-1)[:_P, None]
    c1 = cr1.reshape(-1)[:_P, None]

    e0 = _tc_e0(s0, l0_eW0, l0_eb0, l0_eW1, l0_eb1)
    p0a, p0b = _sc_scatter(e0, row, zeros_sc)
    n1 = _tc_node(x_pad, p0a, p0b, c0, c1, l0_nW0[:1], l0_nW0[1:], l0_nb0,
                  l0_nW1, l0_nb1)

    s1 = _sc_gather(n1[:, 0], row, col).reshape(_N_EDGES, 2)
    e1 = _tc_e1(s1, e0, l1_eW0[:2], l1_eW0[2:], l1_eb0, l1_eW1, l1_eb1)
    p1a, p1b = _sc_scatter(e1, row, zeros_sc)
    n2 = _tc_node(n1, p1a, p1b, c0, c1, l1_nW0[:1], l1_nW0[1:], l1_nb0,
                  l1_nW1, l1_nb1)

    s2 = _sc_gather(n2[:, 0], row, col).reshape(_N_EDGES, 2)
    ev8 = _tc_e2(s2, e1, l2_eW0[:2], l2_eW0[2:], l2_eb0, l2_eW1, l2_eb1)
    out = _sc_finish(ev8.reshape(-1), row, col)
    return out.reshape(2, _N_EDGES)
